# Initial kernel scaffold; baseline (speedup 1.0000x reference)
import jax
import jax.numpy as jnp
from jax.experimental import pallas as pl


def kernel(emb_user, emb_item, src_buys, dst_buys, src_bought_by, dst_bought_by, W0_buys, b0_buys, W0_bb, b0_bb, W1_buys, b1_buys, W1_bb, b1_bb):
    raise NotImplementedError("write your pallas kernel here")


# speedup vs baseline: 2.7807x; 2.7807x over previous
import functools

import jax
import jax.numpy as jnp
from jax import lax
from jax.experimental import pallas as pl
from jax.experimental.pallas import tpu as pltpu
from jax.experimental.pallas import tpu_sc as plsc

N = 50000
E = 256000
D = 128
L = 16
NC = 2
NS = 16
R = 6400
NWIN = 8
NDP = R * NWIN
EPT = E // NS
CH = 2000
NCH = EPT // CH
BK = 128
DUMP = R
TB = 2000
IROWS = E // 128
RPT = 128
IPAD = NS * RPT - IROWS
HZ = NDP // NS
ZR = R // NS

_mesh = plsc.VectorSubcoreMesh(core_axis_name="c", subcore_axis_name="s")
_sc_params = pltpu.CompilerParams(
    needs_layout_passes=False, use_tc_tiling_on_sc=False)


def _deg_body(idx_hbm, zeros_hbm, ones_hbm, deg_hbm, idxb, ones_v, h0, h1, sem):
    c = lax.axis_index("c")
    s = lax.axis_index("s")
    pltpu.sync_copy(ones_hbm, ones_v)
    pltpu.sync_copy(zeros_hbm, h0.at[pl.ds(s * HZ, HZ)])
    pltpu.sync_copy(zeros_hbm, h1.at[pl.ds(s * HZ, HZ)])
    plsc.subcore_barrier()
    for a in range(2):
        hist = h0 if a == 0 else h1
        pltpu.sync_copy(idx_hbm.at[(2 * c + a) * NS + s], idxb)

        def rbody(r, carry, hist=hist):
            pltpu.sync_copy(ones_v, hist.at[idxb.at[r]], add=True)
            return carry

        lax.fori_loop(0, RPT, rbody, 0)
    plsc.subcore_barrier()
    pltpu.sync_copy(h0.at[pl.ds(s * HZ, HZ)],
                    deg_hbm.at[pl.ds((2 * c) * NDP + s * HZ, HZ)])
    pltpu.sync_copy(h1.at[pl.ds(s * HZ, HZ)],
                    deg_hbm.at[pl.ds((2 * c + 1) * NDP + s * HZ, HZ)])


_deg_call = pl.kernel(
    _deg_body,
    out_type=jax.ShapeDtypeStruct((4 * NDP,), jnp.float32),
    mesh=_mesh,
    scratch_types=[
        pltpu.VMEM((RPT, 128), jnp.int32),
        pltpu.VMEM((128,), jnp.float32),
        pltpu.VMEM_SHARED((NDP,), jnp.float32),
        pltpu.VMEM_SHARED((NDP,), jnp.float32),
        pltpu.SemaphoreType.DMA,
    ],
    compiler_params=_sc_params,
)


def _agg_body(p_hbm, src_hbm, dst_hbm, zeros_hbm, out_hbm,
              src_c, dst_c, csrc, cdst, rows_v, acc, sem):
    c = lax.axis_index("c")
    s = lax.axis_index("s")
    ebase = s * EPT
    iota = lax.iota(jnp.int32, L)
    for w in range(NWIN // NC):
        r0 = (w * NC + c) * R
        pltpu.sync_copy(zeros_hbm, acc.at[pl.ds(s * ZR, ZR)])
        plsc.subcore_barrier()

        def chunk_body(ch, mvec, r0=r0):
            pltpu.sync_copy(src_hbm.at[pl.ds(ebase + ch * CH, CH)], src_c)
            pltpu.sync_copy(dst_hbm.at[pl.ds(ebase + ch * CH, CH)], dst_c)

            def vec_body(i, mv, r0=r0):
                sv = src_c[pl.ds(i * L, L)]
                dv = dst_c[pl.ds(i * L, L)]
                dr = dv - r0
                msk = (dr >= 0) & (dr < R)
                pc = plsc.cumsum(msk.astype(jnp.int32))
                pos = mv + pc - 1
                plsc.store_scatter(csrc, [pos >> 7, pos & 127], sv, mask=msk)
                plsc.store_scatter(cdst, [pos >> 7, pos & 127], dr, mask=msk)
                return mv + plsc.all_reduce_population_count(msk)

            return lax.fori_loop(0, CH // L, vec_body, mvec)

        mvec = lax.fori_loop(0, NCH, chunk_body, jnp.zeros((L,), jnp.int32))

        zsrc = jnp.zeros((L,), jnp.int32)
        dmp = jnp.full((L,), DUMP, jnp.int32)
        for j in range(BK // L):
            pp = mvec + j * L + iota
            plsc.store_scatter(csrc, [pp >> 7, pp & 127], zsrc)
            plsc.store_scatter(cdst, [pp >> 7, pp & 127], dmp)
        nb = (jnp.max(mvec) + BK - 1) >> 7

        def blk(j, carry):
            pltpu.async_copy(p_hbm.at[csrc.at[j]], rows_v, sem).wait()
            pltpu.sync_copy(rows_v, acc.at[cdst.at[j]], add=True)
            return carry

        lax.fori_loop(0, nb, blk, 0)
        plsc.subcore_barrier()
        pltpu.sync_copy(acc.at[pl.ds(s * ZR, ZR)],
                        out_hbm.at[pl.ds(r0 + s * ZR, ZR)])


_agg_call = pl.kernel(
    _agg_body,
    out_type=jax.ShapeDtypeStruct((NDP, D), jnp.float32),
    mesh=_mesh,
    scratch_types=[
        pltpu.VMEM((CH,), jnp.int32),
        pltpu.VMEM((CH,), jnp.int32),
        pltpu.VMEM((128, 128), jnp.int32),
        pltpu.VMEM((128, 128), jnp.int32),
        pltpu.VMEM((BK, D), jnp.float32),
        pltpu.VMEM_SHARED((R + 16, D), jnp.float32),
        pltpu.SemaphoreType.DMA,
    ],
    compiler_params=_sc_params,
)


def _proj_body(h_ref, dg_ref, w_ref, o_ref):
    rs = lax.rsqrt(jnp.maximum(dg_ref[...], 1.0))
    o_ref[...] = jnp.dot(h_ref[...] * rs, w_ref[...],
                         preferred_element_type=jnp.float32)


def _proj(h, dgc, w):
    return pl.pallas_call(
        _proj_body,
        grid=(N // TB,),
        in_specs=[
            pl.BlockSpec((TB, D), lambda i: (i, 0)),
            pl.BlockSpec((TB, 1), lambda i: (i, 0)),
            pl.BlockSpec((D, D), lambda i: (0, 0)),
        ],
        out_specs=pl.BlockSpec((TB, D), lambda i: (i, 0)),
        out_shape=jax.ShapeDtypeStruct((N, D), jnp.float32),
    )(h, dgc, w)


def _mid_body(agg_ref, dgi_ref, b_ref, dgo_ref, w_ref, h1_ref, p_ref):
    rs_i = lax.rsqrt(jnp.maximum(dgi_ref[...], 1.0))
    x = agg_ref[...] * rs_i + b_ref[...]
    h1 = jnp.where(x > 0, x, 0.2 * x)
    h1_ref[...] = h1
    rs_o = lax.rsqrt(jnp.maximum(dgo_ref[...], 1.0))
    p_ref[...] = jnp.dot(h1 * rs_o, w_ref[...],
                         preferred_element_type=jnp.float32)


def _mid(agg, dgi, b, dgo, w):
    return pl.pallas_call(
        _mid_body,
        grid=(N // TB,),
        in_specs=[
            pl.BlockSpec((TB, D), lambda i: (i, 0)),
            pl.BlockSpec((TB, 1), lambda i: (i, 0)),
            pl.BlockSpec((1, D), lambda i: (0, 0)),
            pl.BlockSpec((TB, 1), lambda i: (i, 0)),
            pl.BlockSpec((D, D), lambda i: (0, 0)),
        ],
        out_specs=[
            pl.BlockSpec((TB, D), lambda i: (i, 0)),
            pl.BlockSpec((TB, D), lambda i: (i, 0)),
        ],
        out_shape=[
            jax.ShapeDtypeStruct((N, D), jnp.float32),
            jax.ShapeDtypeStruct((N, D), jnp.float32),
        ],
    )(agg, dgi, b, dgo, w)


def _fin_body(agg_ref, dgi_ref, b_ref, res_ref, o_ref):
    rs_i = lax.rsqrt(jnp.maximum(dgi_ref[...], 1.0))
    x = agg_ref[...] * rs_i + b_ref[...]
    o_ref[...] = jnp.where(x > 0, x, 0.2 * x) + res_ref[...]


def _fin(agg, dgi, b, res):
    return pl.pallas_call(
        _fin_body,
        grid=(N // TB,),
        in_specs=[
            pl.BlockSpec((TB, D), lambda i: (i, 0)),
            pl.BlockSpec((TB, 1), lambda i: (i, 0)),
            pl.BlockSpec((1, D), lambda i: (0, 0)),
            pl.BlockSpec((TB, D), lambda i: (i, 0)),
        ],
        out_specs=pl.BlockSpec((TB, D), lambda i: (i, 0)),
        out_shape=jax.ShapeDtypeStruct((N, D), jnp.float32),
    )(agg, dgi, b, res)


def kernel(emb_user, emb_item, src_buys, dst_buys, src_bought_by, dst_bought_by,
           W0_buys, b0_buys, W0_bb, b0_bb, W1_buys, b1_buys, W1_bb, b1_bb):
    idx4 = jnp.stack([src_buys, dst_buys, src_bought_by, dst_bought_by])
    idx4 = idx4.reshape(4, IROWS, 128)
    pad = jnp.full((4, IPAD, 128), NDP - 1, jnp.int32)
    idx4 = jnp.concatenate([idx4, pad], axis=1).reshape(4 * NS, RPT, 128)
    zeros_h = jnp.zeros((HZ,), jnp.float32)
    ones_h = jnp.ones((128,), jnp.float32)
    deg = _deg_call(idx4, zeros_h, ones_h).reshape(4, NDP)

    dg = [deg[a, :N].reshape(N, 1) for a in range(4)]
    zeros_a = jnp.zeros((ZR, D), jnp.float32)
    b0b = b0_buys.reshape(1, D)
    b0bb = b0_bb.reshape(1, D)
    b1b = b1_buys.reshape(1, D)
    b1bb = b1_bb.reshape(1, D)

    p0b = _proj(emb_user, dg[0], W0_buys)
    p0bb = _proj(emb_item, dg[2], W0_bb)
    agg0b = _agg_call(p0b, src_buys, dst_buys, zeros_a)
    agg0bb = _agg_call(p0bb, src_bought_by, dst_bought_by, zeros_a)

    h_i1, p1bb = _mid(agg0b, dg[1], b0b, dg[2], W1_bb)
    h_u1, p1b = _mid(agg0bb, dg[3], b0bb, dg[0], W1_buys)

    agg1b = _agg_call(p1b, src_buys, dst_buys, zeros_a)
    agg1bb = _agg_call(p1bb, src_bought_by, dst_bought_by, zeros_a)

    n_i = _fin(agg1b, dg[1], b1b, h_i1)
    n_u = _fin(agg1bb, dg[3], b1bb, h_u1)
    return (n_u, n_i)


# speedup vs baseline: 2.9852x; 1.0735x over previous
import functools

import jax
import jax.numpy as jnp
from jax import lax
from jax.experimental import pallas as pl
from jax.experimental.pallas import tpu as pltpu
from jax.experimental.pallas import tpu_sc as plsc

N = 50000
E = 256000
D = 128
L = 16
NC = 2
NS = 16
R = 6400
NWIN = 8
NDP = R * NWIN
EPT = E // NS
CH = 2000
NCH = EPT // CH
BK = 128
DUMP = R
TB = 2000
IROWS = E // 128
RPT = 128
IPAD = NS * RPT - IROWS
HZ = NDP // NS
ZR = R // NS

_mesh = plsc.VectorSubcoreMesh(core_axis_name="c", subcore_axis_name="s")
_sc_params = pltpu.CompilerParams(
    needs_layout_passes=False, use_tc_tiling_on_sc=False)


def _deg_body(idx_hbm, zeros_hbm, ones_hbm, deg_hbm, idxb, ones_v, h0, h1, sem):
    c = lax.axis_index("c")
    s = lax.axis_index("s")
    pltpu.sync_copy(ones_hbm, ones_v)
    pltpu.sync_copy(zeros_hbm, h0.at[pl.ds(s * HZ, HZ)])
    pltpu.sync_copy(zeros_hbm, h1.at[pl.ds(s * HZ, HZ)])
    plsc.subcore_barrier()
    for a in range(2):
        hist = h0 if a == 0 else h1
        pltpu.sync_copy(idx_hbm.at[(2 * c + a) * NS + s], idxb)

        def rbody(r, carry, hist=hist):
            pltpu.sync_copy(ones_v, hist.at[idxb.at[r]], add=True)
            return carry

        lax.fori_loop(0, RPT, rbody, 0)
    plsc.subcore_barrier()
    pltpu.sync_copy(h0.at[pl.ds(s * HZ, HZ)],
                    deg_hbm.at[pl.ds((2 * c) * NDP + s * HZ, HZ)])
    pltpu.sync_copy(h1.at[pl.ds(s * HZ, HZ)],
                    deg_hbm.at[pl.ds((2 * c + 1) * NDP + s * HZ, HZ)])


_deg_call = pl.kernel(
    _deg_body,
    out_type=jax.ShapeDtypeStruct((4 * NDP,), jnp.float32),
    mesh=_mesh,
    scratch_types=[
        pltpu.VMEM((RPT, 128), jnp.int32),
        pltpu.VMEM((128,), jnp.float32),
        pltpu.VMEM_SHARED((NDP,), jnp.float32),
        pltpu.VMEM_SHARED((NDP,), jnp.float32),
        pltpu.SemaphoreType.DMA,
    ],
    compiler_params=_sc_params,
)


def _agg_body(p_hbm, src_hbm, dst_hbm, zeros_hbm, out_hbm,
              src_c, dst_c, csrc, cdst, rows_a, rows_b, acc,
              sem_ga, sem_gb):
    c = lax.axis_index("c")
    s = lax.axis_index("s")
    ebase = s * EPT
    iota = lax.iota(jnp.int32, L)
    for w in range(NWIN // NC):
        r0 = (w * NC + c) * R
        pltpu.sync_copy(zeros_hbm, acc.at[pl.ds(s * ZR, ZR)])
        plsc.subcore_barrier()

        def chunk_body(ch, mvec, r0=r0):
            pltpu.sync_copy(src_hbm.at[pl.ds(ebase + ch * CH, CH)], src_c)
            pltpu.sync_copy(dst_hbm.at[pl.ds(ebase + ch * CH, CH)], dst_c)

            def vec_body(i, mv, r0=r0):
                sv = src_c[pl.ds(i * L, L)]
                dv = dst_c[pl.ds(i * L, L)]
                dr = dv - r0
                msk = (dr >= 0) & (dr < R)
                pc = plsc.cumsum(msk.astype(jnp.int32))
                pos = mv + pc - 1
                plsc.store_scatter(csrc, [pos >> 7, pos & 127], sv, mask=msk)
                plsc.store_scatter(cdst, [pos >> 7, pos & 127], dr, mask=msk)
                return mv + plsc.all_reduce_population_count(msk)

            return lax.fori_loop(0, CH // L, vec_body, mvec)

        mvec = lax.fori_loop(0, NCH, chunk_body, jnp.zeros((L,), jnp.int32))

        zsrc = jnp.zeros((L,), jnp.int32)
        dmp = jnp.full((L,), DUMP, jnp.int32)
        for j in range(BK // L):
            pp = mvec + j * L + iota
            plsc.store_scatter(csrc, [pp >> 7, pp & 127], zsrc)
            plsc.store_scatter(cdst, [pp >> 7, pp & 127], dmp)
        nb = (jnp.max(mvec) + BK - 1) >> 7

        @pl.when(nb > 0)
        def _():
            pltpu.async_copy(p_hbm.at[csrc.at[0]], rows_a, sem_ga)

        @pl.when(nb > 1)
        def _():
            pltpu.async_copy(p_hbm.at[csrc.at[1]], rows_b, sem_gb)

        def blk(j, carry):
            even = (j & 1) == 0

            @pl.when(even)
            def _():
                pltpu.make_async_copy(p_hbm.at[csrc.at[j]], rows_a,
                                      sem_ga).wait()
                pltpu.sync_copy(rows_a, acc.at[cdst.at[j]], add=True)

                @pl.when(j + 2 < nb)
                def _():
                    pltpu.async_copy(p_hbm.at[csrc.at[j + 2]], rows_a, sem_ga)

            @pl.when(jnp.logical_not(even))
            def _():
                pltpu.make_async_copy(p_hbm.at[csrc.at[j]], rows_b,
                                      sem_gb).wait()
                pltpu.sync_copy(rows_b, acc.at[cdst.at[j]], add=True)

                @pl.when(j + 2 < nb)
                def _():
                    pltpu.async_copy(p_hbm.at[csrc.at[j + 2]], rows_b, sem_gb)

            return carry

        lax.fori_loop(0, nb, blk, 0)
        plsc.subcore_barrier()
        pltpu.sync_copy(acc.at[pl.ds(s * ZR, ZR)],
                        out_hbm.at[pl.ds(r0 + s * ZR, ZR)])


_agg_call = pl.kernel(
    _agg_body,
    out_type=jax.ShapeDtypeStruct((NDP, D), jnp.float32),
    mesh=_mesh,
    scratch_types=[
        pltpu.VMEM((CH,), jnp.int32),
        pltpu.VMEM((CH,), jnp.int32),
        pltpu.VMEM((128, 128), jnp.int32),
        pltpu.VMEM((128, 128), jnp.int32),
        pltpu.VMEM((BK, D), jnp.float32),
        pltpu.VMEM((BK, D), jnp.float32),
        pltpu.VMEM_SHARED((R + 16, D), jnp.float32),
        pltpu.SemaphoreType.DMA,
        pltpu.SemaphoreType.DMA,
    ],
    compiler_params=_sc_params,
)


def _proj_body(h_ref, dg_ref, w_ref, o_ref):
    rs = lax.rsqrt(jnp.maximum(dg_ref[...], 1.0))
    o_ref[...] = jnp.dot(h_ref[...] * rs, w_ref[...],
                         preferred_element_type=jnp.float32)


def _proj(h, dgc, w):
    return pl.pallas_call(
        _proj_body,
        grid=(N // TB,),
        in_specs=[
            pl.BlockSpec((TB, D), lambda i: (i, 0)),
            pl.BlockSpec((TB, 1), lambda i: (i, 0)),
            pl.BlockSpec((D, D), lambda i: (0, 0)),
        ],
        out_specs=pl.BlockSpec((TB, D), lambda i: (i, 0)),
        out_shape=jax.ShapeDtypeStruct((N, D), jnp.float32),
    )(h, dgc, w)


def _mid_body(agg_ref, dgi_ref, b_ref, dgo_ref, w_ref, h1_ref, p_ref):
    rs_i = lax.rsqrt(jnp.maximum(dgi_ref[...], 1.0))
    x = agg_ref[...] * rs_i + b_ref[...]
    h1 = jnp.where(x > 0, x, 0.2 * x)
    h1_ref[...] = h1
    rs_o = lax.rsqrt(jnp.maximum(dgo_ref[...], 1.0))
    p_ref[...] = jnp.dot(h1 * rs_o, w_ref[...],
                         preferred_element_type=jnp.float32)


def _mid(agg, dgi, b, dgo, w):
    return pl.pallas_call(
        _mid_body,
        grid=(N // TB,),
        in_specs=[
            pl.BlockSpec((TB, D), lambda i: (i, 0)),
            pl.BlockSpec((TB, 1), lambda i: (i, 0)),
            pl.BlockSpec((1, D), lambda i: (0, 0)),
            pl.BlockSpec((TB, 1), lambda i: (i, 0)),
            pl.BlockSpec((D, D), lambda i: (0, 0)),
        ],
        out_specs=[
            pl.BlockSpec((TB, D), lambda i: (i, 0)),
            pl.BlockSpec((TB, D), lambda i: (i, 0)),
        ],
        out_shape=[
            jax.ShapeDtypeStruct((N, D), jnp.float32),
            jax.ShapeDtypeStruct((N, D), jnp.float32),
        ],
    )(agg, dgi, b, dgo, w)


def _fin_body(agg_ref, dgi_ref, b_ref, res_ref, o_ref):
    rs_i = lax.rsqrt(jnp.maximum(dgi_ref[...], 1.0))
    x = agg_ref[...] * rs_i + b_ref[...]
    o_ref[...] = jnp.where(x > 0, x, 0.2 * x) + res_ref[...]


def _fin(agg, dgi, b, res):
    return pl.pallas_call(
        _fin_body,
        grid=(N // TB,),
        in_specs=[
            pl.BlockSpec((TB, D), lambda i: (i, 0)),
            pl.BlockSpec((TB, 1), lambda i: (i, 0)),
            pl.BlockSpec((1, D), lambda i: (0, 0)),
            pl.BlockSpec((TB, D), lambda i: (i, 0)),
        ],
        out_specs=pl.BlockSpec((TB, D), lambda i: (i, 0)),
        out_shape=jax.ShapeDtypeStruct((N, D), jnp.float32),
    )(agg, dgi, b, res)


def kernel(emb_user, emb_item, src_buys, dst_buys, src_bought_by, dst_bought_by,
           W0_buys, b0_buys, W0_bb, b0_bb, W1_buys, b1_buys, W1_bb, b1_bb):
    idx4 = jnp.stack([src_buys, dst_buys, src_bought_by, dst_bought_by])
    idx4 = idx4.reshape(4, IROWS, 128)
    pad = jnp.full((4, IPAD, 128), NDP - 1, jnp.int32)
    idx4 = jnp.concatenate([idx4, pad], axis=1).reshape(4 * NS, RPT, 128)
    zeros_h = jnp.zeros((HZ,), jnp.float32)
    ones_h = jnp.ones((128,), jnp.float32)
    deg = _deg_call(idx4, zeros_h, ones_h).reshape(4, NDP)

    dg = [deg[a, :N].reshape(N, 1) for a in range(4)]
    zeros_a = jnp.zeros((ZR, D), jnp.float32)
    b0b = b0_buys.reshape(1, D)
    b0bb = b0_bb.reshape(1, D)
    b1b = b1_buys.reshape(1, D)
    b1bb = b1_bb.reshape(1, D)

    p0b = _proj(emb_user, dg[0], W0_buys)
    p0bb = _proj(emb_item, dg[2], W0_bb)
    agg0b = _agg_call(p0b, src_buys, dst_buys, zeros_a)
    agg0bb = _agg_call(p0bb, src_bought_by, dst_bought_by, zeros_a)

    h_i1, p1bb = _mid(agg0b, dg[1], b0b, dg[2], W1_bb)
    h_u1, p1b = _mid(agg0bb, dg[3], b0bb, dg[0], W1_buys)

    agg1b = _agg_call(p1b, src_buys, dst_buys, zeros_a)
    agg1bb = _agg_call(p1bb, src_bought_by, dst_bought_by, zeros_a)

    n_i = _fin(agg1b, dg[1], b1b, h_i1)
    n_u = _fin(agg1bb, dg[3], b1bb, h_u1)
    return (n_u, n_i)


# speedup vs baseline: 3.1625x; 1.0594x over previous
import functools

import jax
import jax.numpy as jnp
from jax import lax
from jax.experimental import pallas as pl
from jax.experimental.pallas import tpu as pltpu
from jax.experimental.pallas import tpu_sc as plsc

N = 50000
E = 256000
D = 128
L = 16
NC = 2
NS = 16
R = 6400
NWIN = 8
NDP = R * NWIN
EPT = E // NS
CH = 2000
NCH = EPT // CH
BK = 128
DUMP = R
TB = 2000
IROWS = E // 128
RPT = 128
IPAD = NS * RPT - IROWS
HZ = NDP // NS
ZR = R // NS

_mesh = plsc.VectorSubcoreMesh(core_axis_name="c", subcore_axis_name="s")
_sc_params = pltpu.CompilerParams(
    needs_layout_passes=False, use_tc_tiling_on_sc=False)


def _deg_body(idx_hbm, zeros_hbm, ones_hbm, deg_hbm, idxb, ones_v, h0, h1, sem):
    c = lax.axis_index("c")
    s = lax.axis_index("s")
    pltpu.sync_copy(ones_hbm, ones_v)
    pltpu.sync_copy(zeros_hbm, h0.at[pl.ds(s * HZ, HZ)])
    pltpu.sync_copy(zeros_hbm, h1.at[pl.ds(s * HZ, HZ)])
    plsc.subcore_barrier()
    for a in range(2):
        hist = h0 if a == 0 else h1
        pltpu.sync_copy(idx_hbm.at[(2 * c + a) * NS + s], idxb)

        def rbody(r, carry, hist=hist):
            pltpu.sync_copy(ones_v, hist.at[idxb.at[r]], add=True)
            return carry

        lax.fori_loop(0, RPT, rbody, 0)
    plsc.subcore_barrier()
    pltpu.sync_copy(h0.at[pl.ds(s * HZ, HZ)],
                    deg_hbm.at[pl.ds((2 * c) * NDP + s * HZ, HZ)])
    pltpu.sync_copy(h1.at[pl.ds(s * HZ, HZ)],
                    deg_hbm.at[pl.ds((2 * c + 1) * NDP + s * HZ, HZ)])


_deg_call = pl.kernel(
    _deg_body,
    out_type=jax.ShapeDtypeStruct((4 * NDP,), jnp.float32),
    mesh=_mesh,
    scratch_types=[
        pltpu.VMEM((RPT, 128), jnp.int32),
        pltpu.VMEM((128,), jnp.float32),
        pltpu.VMEM_SHARED((NDP,), jnp.float32),
        pltpu.VMEM_SHARED((NDP,), jnp.float32),
        pltpu.SemaphoreType.DMA,
    ],
    compiler_params=_sc_params,
)


def _agg_body(p_hbm, src_hbm, dst_hbm, zeros_hbm, out_hbm,
              src_c0, dst_c0, src_c1, dst_c1, csrc, cdst, rows_a, rows_b, acc,
              sem_ga, sem_gb, sem_c0, sem_c1):
    c = lax.axis_index("c")
    s = lax.axis_index("s")
    ebase = s * EPT
    iota = lax.iota(jnp.int32, L)
    for w in range(NWIN // NC):
        r0 = (w * NC + c) * R
        pltpu.sync_copy(zeros_hbm, acc.at[pl.ds(s * ZR, ZR)])
        plsc.subcore_barrier()

        bufs = [(src_c0, dst_c0, sem_c0), (src_c1, dst_c1, sem_c1)]

        def _fire(ch):
            sb, db, sm = bufs[ch % 2]
            pltpu.async_copy(src_hbm.at[pl.ds(ebase + ch * CH, CH)], sb, sm)
            pltpu.async_copy(dst_hbm.at[pl.ds(ebase + ch * CH, CH)], db, sm)

        _fire(0)
        mvec = jnp.zeros((L,), jnp.int32)
        for ch in range(NCH):
            if ch + 1 < NCH:
                _fire(ch + 1)
            sb, db, sm = bufs[ch % 2]
            pltpu.make_async_copy(
                src_hbm.at[pl.ds(ebase + ch * CH, CH)], sb, sm).wait()
            pltpu.make_async_copy(
                dst_hbm.at[pl.ds(ebase + ch * CH, CH)], db, sm).wait()

            def vec_body(i, mv, sb=sb, db=db, r0=r0):
                sv = sb[pl.ds(i * L, L)]
                dv = db[pl.ds(i * L, L)]
                dr = dv - r0
                msk = (dr >= 0) & (dr < R)
                pc = plsc.cumsum(msk.astype(jnp.int32))
                pos = mv + pc - 1
                plsc.store_scatter(csrc, [pos >> 7, pos & 127], sv, mask=msk)
                plsc.store_scatter(cdst, [pos >> 7, pos & 127], dr, mask=msk)
                return mv + plsc.all_reduce_population_count(msk)

            mvec = lax.fori_loop(0, CH // L, vec_body, mvec)

        zsrc = jnp.zeros((L,), jnp.int32)
        dmp = jnp.full((L,), DUMP, jnp.int32)
        for j in range(BK // L):
            pp = mvec + j * L + iota
            plsc.store_scatter(csrc, [pp >> 7, pp & 127], zsrc)
            plsc.store_scatter(cdst, [pp >> 7, pp & 127], dmp)
        nb = (jnp.max(mvec) + BK - 1) >> 7

        @pl.when(nb > 0)
        def _():
            pltpu.async_copy(p_hbm.at[csrc.at[0]], rows_a, sem_ga)

        @pl.when(nb > 1)
        def _():
            pltpu.async_copy(p_hbm.at[csrc.at[1]], rows_b, sem_gb)

        def blk(j, carry):
            even = (j & 1) == 0

            @pl.when(even)
            def _():
                pltpu.make_async_copy(p_hbm.at[csrc.at[j]], rows_a,
                                      sem_ga).wait()
                pltpu.sync_copy(rows_a, acc.at[cdst.at[j]], add=True)

                @pl.when(j + 2 < nb)
                def _():
                    pltpu.async_copy(p_hbm.at[csrc.at[j + 2]], rows_a, sem_ga)

            @pl.when(jnp.logical_not(even))
            def _():
                pltpu.make_async_copy(p_hbm.at[csrc.at[j]], rows_b,
                                      sem_gb).wait()
                pltpu.sync_copy(rows_b, acc.at[cdst.at[j]], add=True)

                @pl.when(j + 2 < nb)
                def _():
                    pltpu.async_copy(p_hbm.at[csrc.at[j + 2]], rows_b, sem_gb)

            return carry

        lax.fori_loop(0, nb, blk, 0)
        plsc.subcore_barrier()
        pltpu.sync_copy(acc.at[pl.ds(s * ZR, ZR)],
                        out_hbm.at[pl.ds(r0 + s * ZR, ZR)])


_agg_call = pl.kernel(
    _agg_body,
    out_type=jax.ShapeDtypeStruct((NDP, D), jnp.float32),
    mesh=_mesh,
    scratch_types=[
        pltpu.VMEM((CH,), jnp.int32),
        pltpu.VMEM((CH,), jnp.int32),
        pltpu.VMEM((CH,), jnp.int32),
        pltpu.VMEM((CH,), jnp.int32),
        pltpu.VMEM((128, 128), jnp.int32),
        pltpu.VMEM((128, 128), jnp.int32),
        pltpu.VMEM((BK, D), jnp.float32),
        pltpu.VMEM((BK, D), jnp.float32),
        pltpu.VMEM_SHARED((R + 16, D), jnp.float32),
        pltpu.SemaphoreType.DMA,
        pltpu.SemaphoreType.DMA,
        pltpu.SemaphoreType.DMA,
        pltpu.SemaphoreType.DMA,
    ],
    compiler_params=_sc_params,
)


def _proj_body(h_ref, dg_ref, w_ref, o_ref):
    rs = lax.rsqrt(jnp.maximum(dg_ref[...], 1.0))
    o_ref[...] = jnp.dot(h_ref[...] * rs, w_ref[...],
                         preferred_element_type=jnp.float32)


def _proj(h, dgc, w):
    return pl.pallas_call(
        _proj_body,
        grid=(N // TB,),
        in_specs=[
            pl.BlockSpec((TB, D), lambda i: (i, 0)),
            pl.BlockSpec((TB, 1), lambda i: (i, 0)),
            pl.BlockSpec((D, D), lambda i: (0, 0)),
        ],
        out_specs=pl.BlockSpec((TB, D), lambda i: (i, 0)),
        out_shape=jax.ShapeDtypeStruct((N, D), jnp.float32),
    )(h, dgc, w)


def _mid_body(agg_ref, dgi_ref, b_ref, dgo_ref, w_ref, h1_ref, p_ref):
    rs_i = lax.rsqrt(jnp.maximum(dgi_ref[...], 1.0))
    x = agg_ref[...] * rs_i + b_ref[...]
    h1 = jnp.where(x > 0, x, 0.2 * x)
    h1_ref[...] = h1
    rs_o = lax.rsqrt(jnp.maximum(dgo_ref[...], 1.0))
    p_ref[...] = jnp.dot(h1 * rs_o, w_ref[...],
                         preferred_element_type=jnp.float32)


def _mid(agg, dgi, b, dgo, w):
    return pl.pallas_call(
        _mid_body,
        grid=(N // TB,),
        in_specs=[
            pl.BlockSpec((TB, D), lambda i: (i, 0)),
            pl.BlockSpec((TB, 1), lambda i: (i, 0)),
            pl.BlockSpec((1, D), lambda i: (0, 0)),
            pl.BlockSpec((TB, 1), lambda i: (i, 0)),
            pl.BlockSpec((D, D), lambda i: (0, 0)),
        ],
        out_specs=[
            pl.BlockSpec((TB, D), lambda i: (i, 0)),
            pl.BlockSpec((TB, D), lambda i: (i, 0)),
        ],
        out_shape=[
            jax.ShapeDtypeStruct((N, D), jnp.float32),
            jax.ShapeDtypeStruct((N, D), jnp.float32),
        ],
    )(agg, dgi, b, dgo, w)


def _fin_body(agg_ref, dgi_ref, b_ref, res_ref, o_ref):
    rs_i = lax.rsqrt(jnp.maximum(dgi_ref[...], 1.0))
    x = agg_ref[...] * rs_i + b_ref[...]
    o_ref[...] = jnp.where(x > 0, x, 0.2 * x) + res_ref[...]


def _fin(agg, dgi, b, res):
    return pl.pallas_call(
        _fin_body,
        grid=(N // TB,),
        in_specs=[
            pl.BlockSpec((TB, D), lambda i: (i, 0)),
            pl.BlockSpec((TB, 1), lambda i: (i, 0)),
            pl.BlockSpec((1, D), lambda i: (0, 0)),
            pl.BlockSpec((TB, D), lambda i: (i, 0)),
        ],
        out_specs=pl.BlockSpec((TB, D), lambda i: (i, 0)),
        out_shape=jax.ShapeDtypeStruct((N, D), jnp.float32),
    )(agg, dgi, b, res)


def kernel(emb_user, emb_item, src_buys, dst_buys, src_bought_by, dst_bought_by,
           W0_buys, b0_buys, W0_bb, b0_bb, W1_buys, b1_buys, W1_bb, b1_bb):
    idx4 = jnp.stack([src_buys, dst_buys, src_bought_by, dst_bought_by])
    idx4 = idx4.reshape(4, IROWS, 128)
    pad = jnp.full((4, IPAD, 128), NDP - 1, jnp.int32)
    idx4 = jnp.concatenate([idx4, pad], axis=1).reshape(4 * NS, RPT, 128)
    zeros_h = jnp.zeros((HZ,), jnp.float32)
    ones_h = jnp.ones((128,), jnp.float32)
    deg = _deg_call(idx4, zeros_h, ones_h).reshape(4, NDP)

    dg = [deg[a, :N].reshape(N, 1) for a in range(4)]
    zeros_a = jnp.zeros((ZR, D), jnp.float32)
    b0b = b0_buys.reshape(1, D)
    b0bb = b0_bb.reshape(1, D)
    b1b = b1_buys.reshape(1, D)
    b1bb = b1_bb.reshape(1, D)

    p0b = _proj(emb_user, dg[0], W0_buys)
    p0bb = _proj(emb_item, dg[2], W0_bb)
    agg0b = _agg_call(p0b, src_buys, dst_buys, zeros_a)
    agg0bb = _agg_call(p0bb, src_bought_by, dst_bought_by, zeros_a)

    h_i1, p1bb = _mid(agg0b, dg[1], b0b, dg[2], W1_bb)
    h_u1, p1b = _mid(agg0bb, dg[3], b0bb, dg[0], W1_buys)

    agg1b = _agg_call(p1b, src_buys, dst_buys, zeros_a)
    agg1bb = _agg_call(p1bb, src_bought_by, dst_bought_by, zeros_a)

    n_i = _fin(agg1b, dg[1], b1b, h_i1)
    n_u = _fin(agg1bb, dg[3], b1bb, h_u1)
    return (n_u, n_i)
